# baseline (speedup 1.0000x reference)
import functools

import jax
import jax.numpy as jnp
from jax import lax
from jax.experimental import pallas as pl
from jax.experimental.pallas import tpu as pltpu
from jax.experimental.pallas import tpu_sc as plsc

_D = 64
_B = 4096
_H = 200
_TOTAL = _B * _H
_NC, _NS = 2, 16
_NW = _NC * _NS
_PER_W = _TOTAL // _NW
_CHUNK = 128
_NCHUNK = _PER_W // _CHUNK


def _sc_gather(idx, weights):
    mesh = plsc.VectorSubcoreMesh(core_axis_name="c", subcore_axis_name="s")

    @functools.partial(
        pl.kernel,
        out_type=jax.ShapeDtypeStruct((_TOTAL, _D), jnp.float32),
        mesh=mesh,
        scratch_types=[
            pltpu.VMEM((_NCHUNK, _CHUNK), jnp.int32),
            pltpu.VMEM((_CHUNK, _D), jnp.float32),
            pltpu.SemaphoreType.DMA,
            pltpu.SemaphoreType.DMA,
        ],
        compiler_params=pltpu.CompilerParams(use_tc_tiling_on_sc=False),
    )
    def body(idx_hbm, w_hbm, out_hbm, idx_v, rows_v, gsem, isem):
        wid = lax.axis_index("s") * _NC + lax.axis_index("c")
        pltpu.async_copy(idx_hbm.at[wid], idx_v, isem).wait()
        base = wid * _PER_W

        def step(g, carry):
            pltpu.async_copy(w_hbm.at[idx_v.at[g]], rows_v, gsem).wait()
            pltpu.sync_copy(rows_v, out_hbm.at[pl.ds(base + g * _CHUNK, _CHUNK)])
            return carry

        lax.fori_loop(0, _NCHUNK, step, 0)

    return body(idx, weights)


def kernel(inputs, weights):
    idx = inputs.reshape(_NW, _NCHUNK, _CHUNK)
    out = _sc_gather(idx, weights)
    return out.reshape(_B, _H, _D)


# speedup vs baseline: 1.1163x; 1.1163x over previous
import functools

import jax
import jax.numpy as jnp
from jax import lax
from jax.experimental import pallas as pl
from jax.experimental.pallas import tpu as pltpu
from jax.experimental.pallas import tpu_sc as plsc

_D = 64
_B = 4096
_H = 200
_TOTAL = _B * _H
_NC, _NS = 2, 16
_NW = _NC * _NS
_PER_W = _TOTAL // _NW
_CHUNK = 128
_NCHUNK = _PER_W // _CHUNK
_NBUF = 8
_NGRP = _NCHUNK // _NBUF


def _sc_gather(idx, weights):
    mesh = plsc.VectorSubcoreMesh(core_axis_name="c", subcore_axis_name="s")

    @functools.partial(
        pl.kernel,
        out_type=jax.ShapeDtypeStruct((_TOTAL, _D), jnp.float32),
        mesh=mesh,
        scratch_types=[
            pltpu.VMEM((_NCHUNK, _CHUNK), jnp.int32),
            [pltpu.VMEM((_CHUNK, _D), jnp.float32) for _ in range(_NBUF)],
            [pltpu.SemaphoreType.DMA for _ in range(_NBUF)],
            [pltpu.SemaphoreType.DMA for _ in range(_NBUF)],
            pltpu.SemaphoreType.DMA,
        ],
        compiler_params=pltpu.CompilerParams(use_tc_tiling_on_sc=False),
    )
    def body(idx_hbm, w_hbm, out_hbm, idx_v, rows_v, gsem, wsem, isem):
        wid = lax.axis_index("s") * _NC + lax.axis_index("c")
        pltpu.async_copy(idx_hbm.at[wid], idx_v, isem).wait()
        base = wid * _PER_W

        def gather(chunk, b):
            pltpu.async_copy(w_hbm.at[idx_v.at[chunk]], rows_v[b], gsem[b])

        def writeback(chunk, b):
            pltpu.async_copy(
                rows_v[b], out_hbm.at[pl.ds(base + chunk * _CHUNK, _CHUNK)], wsem[b])

        for b in range(_NBUF):
            gather(b, b)

        def step(g, carry):
            for b in range(_NBUF):
                pltpu.make_async_copy(w_hbm.at[idx_v.at[0]], rows_v[b], gsem[b]).wait()
                writeback(g * _NBUF + b, b)
            for b in range(_NBUF):
                chunk = g * _NBUF + b
                pltpu.make_async_copy(
                    rows_v[b], out_hbm.at[pl.ds(base, _CHUNK)], wsem[b]).wait()

                @pl.when(chunk + _NBUF < _NCHUNK)
                def _():
                    gather(chunk + _NBUF, b)
            return carry

        lax.fori_loop(0, _NGRP, step, 0)

    return body(idx, weights)


def kernel(inputs, weights):
    idx = inputs.reshape(_NW, _NCHUNK, _CHUNK)
    out = _sc_gather(idx, weights)
    return out.reshape(_B, _H, _D)
